# baseline (speedup 1.0000x reference)
import functools

import jax
import jax.numpy as jnp
from jax import lax
from jax.experimental import pallas as pl
from jax.experimental.pallas import tpu as pltpu
from jax.experimental.pallas import tpu_sc as plsc

NUM_EMB = 1_000_000
DIM = 64
BATCH = 16384
HIST = 50
HIST_PAD = 56

NC = 2
NS = 16
NW = NC * NS

BTILE = 128
NBB = BATCH // BTILE
NBLOCKS = HIST * NBB
PER_W = NBLOCKS // NW


def _mesh():
    return plsc.VectorSubcoreMesh(core_axis_name="c", subcore_axis_name="s")


@functools.partial(
    pl.kernel,
    out_type=jax.ShapeDtypeStruct((HIST, DIM // 8, NBB, 8, BTILE),
                                  jnp.float32),
    mesh=_mesh(),
    compiler_params=pltpu.CompilerParams(use_tc_tiling_on_sc=False,
                                         needs_layout_passes=False),
    scratch_types=[
        pltpu.VMEM((BTILE,), jnp.int32),
        pltpu.VMEM((BTILE,), jnp.int32),
        pltpu.VMEM((BTILE, DIM), jnp.float32),
        pltpu.VMEM((BTILE, DIM), jnp.float32),
        pltpu.VMEM((DIM // 8, 8, BTILE), jnp.float32),
        pltpu.VMEM((DIM // 8, 8, BTILE), jnp.float32),
        pltpu.SemaphoreType.DMA,
        pltpu.SemaphoreType.DMA,
        pltpu.SemaphoreType.DMA,
        pltpu.SemaphoreType.DMA,
        pltpu.SemaphoreType.DMA,
        pltpu.SemaphoreType.DMA,
    ],
)
def _gather_kernel(idx_hbm, table_hbm, out_hbm, i0, i1, rows0, rows1, t0, t1,
                   i0sem, i1sem, g0sem, g1sem, o0sem, o1sem):
    wid = lax.axis_index("s") * NC + lax.axis_index("c")
    base = wid * PER_W

    ibuf = (i0, i1)
    rows = (rows0, rows1)
    tbuf = (t0, t1)
    isem = (i0sem, i1sem)
    gsem = (g0sem, g1sem)
    osem = (o0sem, o1sem)

    i16 = lax.iota(jnp.int32, 16)
    perm = [(i16 + d) & 15 for d in range(16)]

    def fire_idx(i, slot):
        g = base + i
        th = g // NBB
        bb = g % NBB
        pltpu.async_copy(idx_hbm.at[th // 8, bb, th % 8, :], ibuf[slot],
                         isem[slot])

    def drain_idx(slot):
        pltpu.make_async_copy(idx_hbm.at[0, 0, 0, :], ibuf[slot],
                              isem[slot]).wait()

    def fire_gather(slot):
        pltpu.async_copy(table_hbm.at[ibuf[slot]], rows[slot], gsem[slot])

    def drain(sem, slot):
        pltpu.make_async_copy(table_hbm.at[pl.ds(0, BTILE)], rows[slot],
                              sem).wait()

    def transpose(slot):
        r = rows[slot]
        t = tbuf[slot]

        @plsc.parallel_loop(0, BTILE // 16, unroll=2)
        def _(bblk):
            bvec = i16 + bblk * 16
            for cb in range(DIM // 16):
                for d in range(16):
                    cvec = perm[d] + (cb * 16)
                    v = plsc.load_gather(r, [bvec, cvec])
                    plsc.store_scatter(
                        t, [cvec >> 3, cvec & 7, bvec], v)

    def fire_out(i, slot):
        g = base + i
        th = g // NBB
        bb = g % NBB
        pltpu.async_copy(tbuf[slot], out_hbm.at[th, :, bb, :, :], osem[slot])

    fire_idx(0, 0)
    fire_idx(1, 1)
    drain_idx(0)
    fire_gather(0)
    drain_idx(1)
    fire_gather(1)

    @pl.loop(0, PER_W, step=2)
    def _(i):
        for s in (0, 1):
            ii = i + s
            drain(gsem[s], s)

            @pl.when(ii + 2 < PER_W)
            def _():
                fire_idx(ii + 2, s)

            @pl.when(ii >= 2)
            def _():
                drain(osem[s], s)

            transpose(s)
            fire_out(ii, s)

            @pl.when(ii + 2 < PER_W)
            def _():
                drain_idx(s)
                fire_gather(s)

    drain(osem[0], 0)
    drain(osem[1], 1)


def kernel(token_ids, W):
    tp = jnp.pad(token_ids.astype(jnp.int32).T,
                 ((0, HIST_PAD - HIST), (0, 0)))
    idx4 = tp.reshape(HIST_PAD // 8, 8, NBB, BTILE).transpose(0, 2, 1, 3)
    x = _gather_kernel(idx4, W)
    return x.transpose(2, 4, 0, 1, 3).reshape(BATCH, HIST, DIM)


# speedup vs baseline: 1.0206x; 1.0206x over previous
import functools

import jax
import jax.numpy as jnp
from jax import lax
from jax.experimental import pallas as pl
from jax.experimental.pallas import tpu as pltpu
from jax.experimental.pallas import tpu_sc as plsc

NUM_EMB = 1_000_000
DIM = 64
BATCH = 16384
HIST = 50
HIST_PAD = 56

NC = 2
NS = 16
NW = NC * NS

BTILE = 128
NBB = BATCH // BTILE
NBLOCKS = HIST * NBB
PER_W = NBLOCKS // NW


def _mesh():
    return plsc.VectorSubcoreMesh(core_axis_name="c", subcore_axis_name="s")


@functools.partial(
    pl.kernel,
    out_type=jax.ShapeDtypeStruct((HIST, DIM // 8, NBB, 8, BTILE),
                                  jnp.float32),
    mesh=_mesh(),
    compiler_params=pltpu.CompilerParams(use_tc_tiling_on_sc=False,
                                         needs_layout_passes=False),
    scratch_types=[
        pltpu.VMEM((HIST_PAD // 8, NBB // NW, 8, BTILE), jnp.int32),
        pltpu.VMEM((BTILE, DIM), jnp.float32),
        pltpu.VMEM((BTILE, DIM), jnp.float32),
        pltpu.VMEM((DIM // 8, 8, BTILE), jnp.float32),
        pltpu.VMEM((DIM // 8, 8, BTILE), jnp.float32),
        pltpu.SemaphoreType.DMA,
        pltpu.SemaphoreType.DMA,
        pltpu.SemaphoreType.DMA,
        pltpu.SemaphoreType.DMA,
    ],
)
def _gather_kernel(idx_hbm, table_hbm, out_hbm, idx_all, rows0, rows1, t0, t1,
                   g0sem, g1sem, o0sem, o1sem):
    wid = lax.axis_index("s") * NC + lax.axis_index("c")
    bpw = NBB // NW
    pltpu.sync_copy(idx_hbm.at[:, pl.ds(wid * bpw, bpw), :, :], idx_all)

    rows = (rows0, rows1)
    tbuf = (t0, t1)
    gsem = (g0sem, g1sem)
    osem = (o0sem, o1sem)

    i16 = lax.iota(jnp.int32, 16)
    perm = [(i16 + d) & 15 for d in range(16)]

    def fire_gather(i, slot):
        th = i // bpw
        j = i % bpw
        pltpu.async_copy(table_hbm.at[idx_all.at[th // 8, j, th % 8, :]],
                         rows[slot], gsem[slot])

    def drain(sem, slot):
        pltpu.make_async_copy(table_hbm.at[pl.ds(0, BTILE)], rows[slot],
                              sem).wait()

    def transpose(slot):
        r = rows[slot]
        t = tbuf[slot]

        @plsc.parallel_loop(0, BTILE // 16, unroll=2)
        def _(bblk):
            bvec = i16 + bblk * 16
            for cb in range(DIM // 16):
                for d in range(16):
                    cvec = perm[d] + (cb * 16)
                    v = plsc.load_gather(r, [bvec, cvec])
                    plsc.store_scatter(
                        t, [cvec >> 3, cvec & 7, bvec], v)

    def fire_out(i, slot):
        th = i // bpw
        bb = wid * bpw + (i % bpw)
        pltpu.async_copy(tbuf[slot], out_hbm.at[th, :, bb, :, :], osem[slot])

    fire_gather(0, 0)
    fire_gather(1, 1)

    @pl.loop(0, PER_W, step=2)
    def _(i):
        for s in (0, 1):
            ii = i + s
            drain(gsem[s], s)

            @pl.when(ii >= 2)
            def _():
                drain(osem[s], s)

            transpose(s)
            fire_out(ii, s)

            @pl.when(ii + 2 < PER_W)
            def _():
                fire_gather(ii + 2, s)

    drain(osem[0], 0)
    drain(osem[1], 1)


def kernel(token_ids, W):
    tp = jnp.pad(token_ids.astype(jnp.int32).T,
                 ((0, HIST_PAD - HIST), (0, 0)))
    idx4 = tp.reshape(HIST_PAD // 8, 8, NBB, BTILE).transpose(0, 2, 1, 3)
    x = _gather_kernel(idx4, W)
    return x.transpose(2, 4, 0, 1, 3).reshape(BATCH, HIST, DIM)


# speedup vs baseline: 1.2456x; 1.2205x over previous
import functools

import jax
import jax.numpy as jnp
from jax import lax
from jax.experimental import pallas as pl
from jax.experimental.pallas import tpu as pltpu
from jax.experimental.pallas import tpu_sc as plsc

NUM_EMB = 1_000_000
DIM = 64
BATCH = 16384
HIST = 50
HIST_PAD = 56

NC = 2
NS = 16
NW = NC * NS

BTILE = 128
NBB = BATCH // BTILE
NBLOCKS = HIST * NBB
PER_W = NBLOCKS // NW


def _mesh():
    return plsc.VectorSubcoreMesh(core_axis_name="c", subcore_axis_name="s")


@functools.partial(
    pl.kernel,
    out_type=jax.ShapeDtypeStruct((HIST, DIM // 8, NBB, 8, BTILE),
                                  jnp.float32),
    mesh=_mesh(),
    compiler_params=pltpu.CompilerParams(use_tc_tiling_on_sc=False,
                                         needs_layout_passes=False),
    scratch_types=[
        pltpu.VMEM((HIST_PAD // 8, NBB // NW, 8, BTILE), jnp.int32),
        pltpu.VMEM((BTILE, DIM), jnp.float32),
        pltpu.VMEM((BTILE, DIM), jnp.float32),
        pltpu.VMEM((DIM // 8, 8, BTILE), jnp.float32),
        pltpu.VMEM((DIM // 8, 8, BTILE), jnp.float32),
        pltpu.SemaphoreType.DMA,
        pltpu.SemaphoreType.DMA,
        pltpu.SemaphoreType.DMA,
        pltpu.SemaphoreType.DMA,
    ],
)
def _gather_kernel(idx_hbm, table_hbm, out_hbm, idx_all, rows0, rows1, t0, t1,
                   g0sem, g1sem, o0sem, o1sem):
    wid = lax.axis_index("s") * NC + lax.axis_index("c")
    bpw = NBB // NW
    pltpu.sync_copy(idx_hbm.at[:, pl.ds(wid * bpw, bpw), :, :], idx_all)

    rows = (rows0, rows1)
    tbuf = (t0, t1)
    gsem = (g0sem, g1sem)
    osem = (o0sem, o1sem)

    i16 = lax.iota(jnp.int32, 16)
    perm = [(i16 + d) & 15 for d in range(16)]

    def fire_gather(i, slot):
        th = i // bpw
        j = i % bpw
        pltpu.async_copy(table_hbm.at[idx_all.at[th // 8, j, th % 8, :]],
                         rows[slot], gsem[slot])

    def drain(sem, slot):
        pltpu.make_async_copy(table_hbm.at[pl.ds(0, BTILE)], rows[slot],
                              sem).wait()

    def transpose(slot):
        r = rows[slot]
        t = tbuf[slot]

        @plsc.parallel_loop(0, BTILE // 16, unroll=4)
        def _(bblk):
            bvec = i16 + bblk * 16
            for cb in range(DIM // 16):
                for d in range(16):
                    cvec = perm[d] + (cb * 16)
                    v = plsc.load_gather(r, [bvec, cvec])
                    plsc.store_scatter(
                        t, [cvec >> 3, cvec & 7, bvec], v)

    def fire_out(i, slot):
        th = i // bpw
        bb = wid * bpw + (i % bpw)
        pltpu.async_copy(tbuf[slot], out_hbm.at[th, :, bb, :, :], osem[slot])

    fire_gather(0, 0)
    fire_gather(1, 1)

    @pl.loop(0, PER_W, step=2)
    def _(i):
        for s in (0, 1):
            ii = i + s
            drain(gsem[s], s)

            @pl.when(ii >= 2)
            def _():
                drain(osem[s], s)

            transpose(s)
            fire_out(ii, s)

            @pl.when(ii + 2 < PER_W)
            def _():
                fire_gather(ii + 2, s)

    drain(osem[0], 0)
    drain(osem[1], 1)


def kernel(token_ids, W):
    tp = jnp.pad(token_ids.astype(jnp.int32).T,
                 ((0, HIST_PAD - HIST), (0, 0)))
    idx4 = tp.reshape(HIST_PAD // 8, 8, NBB, BTILE).transpose(0, 2, 1, 3)
    x = _gather_kernel(idx4, W)
    return x.transpose(2, 4, 0, 1, 3).reshape(BATCH, HIST, DIM)
